# Initial kernel scaffold; baseline (speedup 1.0000x reference)
import jax
import jax.numpy as jnp
from jax.experimental import pallas as pl


def kernel(x, edge_index, W0, b0, W1, b1, W2, b2, W3, b3):
    raise NotImplementedError("write your pallas kernel here")


# speedup vs baseline: 13.7054x; 13.7054x over previous
import functools

import jax
import jax.numpy as jnp
from jax import lax
from jax.experimental import pallas as pl
from jax.experimental.pallas import tpu as pltpu
from jax.experimental.pallas import tpu_sc as plsc

N = 10000
NPAD = 10240
E = 320000
D = 128
DOUT = 64

NC = 2
NS = 16
NW = NC * NS
EPT = E // NW
B = 80
NB = EPT // B
RPS = NPAD // NS
DEGW = 128

_MESH = plsc.VectorSubcoreMesh(core_axis_name="c", subcore_axis_name="s",
                               num_cores=NC, num_subcores=NS)


def _make_edge_scatter(width):
  @functools.partial(
      pl.kernel,
      out_type=jax.ShapeDtypeStruct((NC, NPAD, width), jnp.float32),
      mesh=_MESH,
      scratch_types=[
          pltpu.VMEM((NB, B), jnp.int32),
          pltpu.VMEM((NB, B), jnp.int32),
          pltpu.VMEM((B, width), jnp.float32),
          pltpu.VMEM_SHARED((NPAD, width), jnp.float32),
          pltpu.SemaphoreType.DMA,
      ],
  )
  def scat(g_hbm, src_hbm, dst_hbm, zeros_hbm, out_hbm,
           src_v, dst_v, rows_v, acc, sem):
    cid = lax.axis_index("c")
    sid = lax.axis_index("s")
    wid = cid * NS + sid
    pltpu.sync_copy(zeros_hbm, acc.at[pl.ds(sid * RPS, RPS)])
    pltpu.sync_copy(src_hbm.at[wid], src_v)
    pltpu.sync_copy(dst_hbm.at[wid], dst_v)
    plsc.subcore_barrier()

    def body(j, carry):
      pltpu.async_copy(g_hbm.at[src_v.at[j]], rows_v, sem).wait()
      pltpu.sync_copy(rows_v, acc.at[dst_v.at[j]], add=True)
      return carry

    lax.fori_loop(0, NB, body, 0)
    plsc.subcore_barrier()
    pltpu.sync_copy(acc.at[pl.ds(sid * RPS, RPS)],
                    out_hbm.at[cid, pl.ds(sid * RPS, RPS)])

  return scat


@functools.partial(
    pl.kernel,
    out_type=jax.ShapeDtypeStruct((NC, NPAD, DEGW), jnp.float32),
    mesh=_MESH,
    scratch_types=[
        pltpu.VMEM((NB, B), jnp.int32),
        pltpu.VMEM((B, DEGW), jnp.float32),
        pltpu.VMEM_SHARED((NPAD, DEGW), jnp.float32),
    ],
)
def _degree_kernel(ones_hbm, dst_hbm, zeros_hbm, out_hbm,
                   dst_v, ones_v, acc):
  cid = lax.axis_index("c")
  sid = lax.axis_index("s")
  wid = cid * NS + sid
  pltpu.sync_copy(zeros_hbm, acc.at[pl.ds(sid * RPS, RPS)])
  pltpu.sync_copy(dst_hbm.at[wid], dst_v)
  pltpu.sync_copy(ones_hbm, ones_v)
  plsc.subcore_barrier()

  def body(j, carry):
    pltpu.sync_copy(ones_v, acc.at[dst_v.at[j]], add=True)
    return carry

  lax.fori_loop(0, NB, body, 0)
  plsc.subcore_barrier()
  pltpu.sync_copy(acc.at[pl.ds(sid * RPS, RPS)],
                  out_hbm.at[cid, pl.ds(sid * RPS, RPS)])


GRID = 8
BR = NPAD // GRID

_row = lambda w: pl.BlockSpec((BR, w), lambda i: (i, 0))
_full = lambda r, w: pl.BlockSpec((r, w), lambda i: (0, 0))


def _stage_a(x_ref, p0_ref, p1_ref, w_ref, g_ref, dis_ref):
  dis = lax.rsqrt(1.0 + p0_ref[...] + p1_ref[...])
  g_ref[...] = jnp.dot(x_ref[...], w_ref[...],
                       preferred_element_type=jnp.float32) * dis
  dis_ref[...] = dis


def _stage_mid(s0_ref, s1_ref, g_ref, dis_ref, b_ref, w_ref, out_ref):
  dis = dis_ref[...]
  h = dis * (s0_ref[...] + s1_ref[...] + g_ref[...]) + b_ref[...]
  h = jnp.maximum(h, 0.0)
  out_ref[...] = jnp.dot(h, w_ref[...],
                         preferred_element_type=jnp.float32) * dis


def _stage_out(s0_ref, s1_ref, g_ref, dis_ref, b_ref, out_ref):
  o = dis_ref[...] * (s0_ref[...] + s1_ref[...] + g_ref[...]) + b_ref[...]
  m = jnp.max(o, axis=1, keepdims=True)
  e = o - m
  out_ref[...] = e - jnp.log(jnp.sum(jnp.exp(e), axis=1, keepdims=True))


def _tc_a(x, p0, p1, w):
  return pl.pallas_call(
      _stage_a,
      grid=(GRID,),
      in_specs=[_row(D), _row(1), _row(1), _full(D, D)],
      out_specs=[_row(D), _row(1)],
      out_shape=[jax.ShapeDtypeStruct((NPAD, D), jnp.float32),
                 jax.ShapeDtypeStruct((NPAD, 1), jnp.float32)],
  )(x, p0, p1, w)


def _tc_mid(s0, s1, g, dis, b, w, dout):
  return pl.pallas_call(
      _stage_mid,
      grid=(GRID,),
      in_specs=[_row(D), _row(D), _row(D), _row(1), _full(1, D), _full(D, dout)],
      out_specs=_row(dout),
      out_shape=jax.ShapeDtypeStruct((NPAD, dout), jnp.float32),
  )(s0, s1, g, dis, b, w)


def _tc_out(s0, s1, g, dis, b):
  return pl.pallas_call(
      _stage_out,
      grid=(GRID,),
      in_specs=[_row(DOUT), _row(DOUT), _row(DOUT), _row(1), _full(1, DOUT)],
      out_specs=_row(DOUT),
      out_shape=jax.ShapeDtypeStruct((NPAD, DOUT), jnp.float32),
  )(s0, s1, g, dis, b)


_scatter_d = _make_edge_scatter(D)


def kernel(x, edge_index, W0, b0, W1, b1, W2, b2, W3, b3):
  src = edge_index[0].reshape(NW, NB, B)
  dst = edge_index[1].reshape(NW, NB, B)

  x_pad = jnp.pad(x, ((0, NPAD - N), (0, 0)))
  zeros_d = jnp.zeros((RPS, D), jnp.float32)
  zeros_degw = jnp.zeros((RPS, DEGW), jnp.float32)
  ones_deg = jnp.ones((B, DEGW), jnp.float32)

  deg = _degree_kernel(ones_deg, dst, zeros_degw)
  p0 = deg[0, :, 0:1]
  p1 = deg[1, :, 0:1]

  g0, dis = _tc_a(x_pad, p0, p1, W0)

  s = _scatter_d(g0, src, dst, zeros_d)
  g1 = _tc_mid(s[0], s[1], g0, dis, b0.reshape(1, D), W1, D)

  s = _scatter_d(g1, src, dst, zeros_d)
  g2 = _tc_mid(s[0], s[1], g1, dis, b1.reshape(1, D), W2, D)

  s = _scatter_d(g2, src, dst, zeros_d)
  g3 = _tc_mid(s[0], s[1], g2, dis, b2.reshape(1, D), W3, DOUT)

  g3p = jnp.pad(g3, ((0, 0), (0, D - DOUT)))
  s = _scatter_d(g3p, src, dst, zeros_d)
  out = _tc_out(s[0, :, :DOUT], s[1, :, :DOUT], g3, dis, b3.reshape(1, DOUT))

  return out[:N]
